# Initial kernel scaffold; baseline (speedup 1.0000x reference)
import jax
import jax.numpy as jnp
from jax.experimental import pallas as pl


def kernel(x):
    raise NotImplementedError("write your pallas kernel here")


# speedup vs baseline: 1.0098x; 1.0098x over previous
import functools

import jax
import jax.numpy as jnp
from jax.experimental import pallas as pl
from jax.experimental.pallas import tpu as pltpu

_KS0 = 0x6D3E048F
_KS1 = 0x1022172D
_KS2 = _KS0 ^ _KS1 ^ 0x1BD11BDA

_T_HI = float(jnp.float32(0.003638321))
_T_LO = float(jnp.float32(0.003336203))
_SALT = float(jnp.float32(0.3890121))
_PEPPER = float(jnp.float32(-0.2562604))

_ROTATIONS = ((13, 15, 26, 6), (17, 29, 16, 24))

_LANE = 384
_ROWS = 64 * 3 * 384


def _threefry_bits(idx):
    ks = (jnp.uint32(_KS0), jnp.uint32(_KS1), jnp.uint32(_KS2))
    x0 = jnp.full_like(idx, ks[0])
    x1 = idx + ks[1]
    for g in range(5):
        for r in _ROTATIONS[g % 2]:
            x0 = x0 + x1
            x1 = (x1 << jnp.uint32(r)) | (x1 >> jnp.uint32(32 - r))
            x1 = x0 ^ x1
        x0 = x0 + ks[(g + 1) % 3]
        x1 = x1 + ks[(g + 2) % 3] + jnp.uint32(g + 1)
    return x0 ^ x1


def _body(x_ref, o_ref, *, block_rows):
    base_row = pl.program_id(0) * block_rows
    shape = (block_rows, _LANE)
    row = jax.lax.broadcasted_iota(jnp.uint32, shape, 0)
    col = jax.lax.broadcasted_iota(jnp.uint32, shape, 1)
    idx = (jnp.uint32(base_row) + row) * jnp.uint32(_LANE) + col
    bits = _threefry_bits(idx)
    mant = (bits >> jnp.uint32(9)) | jnp.uint32(0x3F800000)
    noise = pltpu.bitcast(mant, jnp.float32) - jnp.float32(1.0)
    x = x_ref[...]
    out = jnp.where(noise >= jnp.float32(1.0 - _T_HI), jnp.float32(_SALT), x)
    out = jnp.where(noise <= jnp.float32(_T_LO), jnp.float32(_PEPPER), out)
    o_ref[...] = out


@jax.jit
def kernel(x):
    block_rows = 1024
    grid = _ROWS // block_rows
    x2 = x.reshape(_ROWS, _LANE)
    out = pl.pallas_call(
        functools.partial(_body, block_rows=block_rows),
        grid=(grid,),
        in_specs=[pl.BlockSpec((block_rows, _LANE), lambda i: (i, 0))],
        out_specs=pl.BlockSpec((block_rows, _LANE), lambda i: (i, 0)),
        out_shape=jax.ShapeDtypeStruct((_ROWS, _LANE), jnp.float32),
        compiler_params=pltpu.CompilerParams(
            dimension_semantics=("arbitrary",),
        ),
    )(x2)
    return out.reshape(x.shape)
